# Initial kernel scaffold; baseline (speedup 1.0000x reference)
import jax
import jax.numpy as jnp
from jax.experimental import pallas as pl


def kernel(prediction):
    raise NotImplementedError("write your pallas kernel here")


# speedup vs baseline: 11.1598x; 11.1598x over previous
import functools

import jax
import jax.numpy as jnp
from jax import lax
from jax.experimental import pallas as pl
from jax.experimental.pallas import tpu as pltpu
from jax.experimental.pallas import tpu_sc as plsc

IMG_H = 512.0
IMG_W = 512.0
CONF_THRESH = 0.01
IOU_THRESH = 0.45
TOP_K = 200
NMS_MAX = 400
CAND = 512
NCLS = 80


def _nms_body(s_ref, c12_ref, out_ref,
              x1_s, y1_s, x2_s, y2_s, ar_s, sup_s, kept_s,
              c0_s, c1_s, c2_s, c3_s, c4_s, c5_s):
    s = s_ref[0]
    l0 = c12_ref[0, 0]
    l1 = c12_ref[0, 1]
    l2 = c12_ref[0, 2]
    l3 = c12_ref[0, 3]
    a0 = c12_ref[0, 4]
    a1 = c12_ref[0, 5]
    a2 = c12_ref[0, 6]
    a3 = c12_ref[0, 7]
    v0 = c12_ref[0, 8]
    v1 = c12_ref[0, 9]
    v2 = c12_ref[0, 10]
    v3 = c12_ref[0, 11]

    cx = l0 * v0 * a2 + a0
    cy = l1 * v1 * a3 + a1
    w = jnp.exp(l2 * v2) * a2
    h = jnp.exp(l3 * v3) * a3
    x1 = (cx - 0.5 * w) * IMG_W
    y1 = (cy - 0.5 * h) * IMG_H
    x2 = (cx + 0.5 * w) * IMG_W
    y2 = (cy + 0.5 * h) * IMG_H
    x1_s[...] = x1
    y1_s[...] = y1
    x2_s[...] = x2
    y2_s[...] = y2
    ar_s[...] = (x2 - x1) * (y2 - y1)
    sup_s[...] = jnp.zeros((CAND, NCLS), jnp.float32)

    def step(i, _):
        bx1 = x1_s[pl.ds(i, 1), :]
        by1 = y1_s[pl.ds(i, 1), :]
        bx2 = x2_s[pl.ds(i, 1), :]
        by2 = y2_s[pl.ds(i, 1), :]
        bar = ar_s[pl.ds(i, 1), :]
        bsup = sup_s[pl.ds(i, 1), :]
        bs = s_ref[0, pl.ds(i, 1), :]
        keep = (bs > 0.0) & (bsup < 0.5)
        kept_s[pl.ds(i, 1), :] = jnp.where(keep, 1.0, 0.0)
        ix1 = jnp.maximum(x1_s[...], bx1)
        iy1 = jnp.maximum(y1_s[...], by1)
        ix2 = jnp.minimum(x2_s[...], bx2)
        iy2 = jnp.minimum(y2_s[...], by2)
        inter = jnp.maximum(ix2 - ix1, 0.0) * jnp.maximum(iy2 - iy1, 0.0)
        union = jnp.maximum(ar_s[...] + bar - inter, 1e-8)
        cond = inter > IOU_THRESH * union
        sup_s[...] = jnp.maximum(sup_s[...], jnp.where(cond & keep, 1.0, 0.0))
        return 0

    lax.fori_loop(0, CAND, step, 0)

    kept = kept_s[...]
    pos = kept
    sh = 1
    while sh < CAND:
        pos = pos + jnp.pad(pos[:-sh, :], ((sh, 0), (0, 0)))
        sh *= 2
    masked = jnp.where((kept > 0.5) & (pos <= float(NMS_MAX)), s, 0.0)

    rank_iota = lax.broadcasted_iota(jnp.int32, (CAND, NCLS), 0)
    cls_iota = lax.broadcasted_iota(jnp.int32, (CAND, NCLS), 1)
    key_iota = cls_iota * CAND + rank_iota
    BIG = jnp.int32(2 ** 30)

    def pick(k, m_arr):
        m = jnp.max(m_arr)
        eq = m_arr == m
        keym = jnp.where(eq, key_iota, BIG)
        kmin = jnp.min(keym)
        onehot = keym == kmin
        clsi = kmin // CAND
        ok = m > 0.0
        x1v = jnp.sum(jnp.where(onehot, x1_s[...], 0.0))
        y1v = jnp.sum(jnp.where(onehot, y1_s[...], 0.0))
        x2v = jnp.sum(jnp.where(onehot, x2_s[...], 0.0))
        y2v = jnp.sum(jnp.where(onehot, y2_s[...], 0.0))
        zero = jnp.float32(0.0)
        c0_s[pl.ds(k, 1), :] = jnp.where(ok, (clsi + 1).astype(jnp.float32), zero).reshape(1, 1)
        c1_s[pl.ds(k, 1), :] = jnp.where(ok, m, zero).reshape(1, 1)
        c2_s[pl.ds(k, 1), :] = jnp.where(ok, x1v, zero).reshape(1, 1)
        c3_s[pl.ds(k, 1), :] = jnp.where(ok, y1v, zero).reshape(1, 1)
        c4_s[pl.ds(k, 1), :] = jnp.where(ok, x2v, zero).reshape(1, 1)
        c5_s[pl.ds(k, 1), :] = jnp.where(ok, y2v, zero).reshape(1, 1)
        return jnp.where(onehot, -1.0, m_arr)

    lax.fori_loop(0, TOP_K, pick, masked)

    out_ref[0] = jnp.concatenate(
        [c0_s[...], c1_s[...], c2_s[...], c3_s[...], c4_s[...], c5_s[...]],
        axis=1)


def _nms_pallas(s_in, c12_in):
    B = s_in.shape[0]
    f32 = jnp.float32
    scr = [pltpu.VMEM((CAND, NCLS), f32) for _ in range(7)]
    scr += [pltpu.VMEM((TOP_K, 1), f32) for _ in range(6)]
    return pl.pallas_call(
        _nms_body,
        grid=(B,),
        in_specs=[
            pl.BlockSpec((1, CAND, NCLS), lambda b: (b, 0, 0)),
            pl.BlockSpec((1, 12, CAND, NCLS), lambda b: (b, 0, 0, 0)),
        ],
        out_specs=pl.BlockSpec((1, TOP_K, 6), lambda b: (b, 0, 0)),
        out_shape=jax.ShapeDtypeStruct((B, TOP_K, 6), f32),
        scratch_shapes=scr,
    )(s_in, c12_in)


N_BOX = 20000
NINST = 8 * NCLS
NWORK = 32
PER_W = NINST // NWORK
NVEC = N_BOX // 16
NBKT = 256
RADIX = 32


def _sc_topk(scoresT, pred12):
    f32, i32, u32 = jnp.float32, jnp.int32, jnp.uint32
    mesh = plsc.VectorSubcoreMesh(core_axis_name="c", subcore_axis_name="s")
    iota16 = lambda: lax.broadcasted_iota(i32, (16,), 0)

    @functools.partial(
        pl.kernel,
        mesh=mesh,
        out_type=[
            jax.ShapeDtypeStruct((NINST, CAND), f32),
            jax.ShapeDtypeStruct((NINST, CAND, 12), f32),
        ],
        scratch_types=[
            pltpu.VMEM((N_BOX,), f32),
            pltpu.VMEM((NBKT,), i32),
            pltpu.VMEM((N_BOX,), u32),
            pltpu.VMEM((N_BOX,), i32),
            pltpu.VMEM((N_BOX,), u32),
            pltpu.VMEM((N_BOX,), i32),
            pltpu.VMEM((RADIX,), i32),
            pltpu.VMEM((CAND,), f32),
            pltpu.VMEM((CAND,), i32),
            pltpu.VMEM((CAND, 12), f32),
            pltpu.SemaphoreType.DMA,
        ],
    )
    def sc_body(scores_hbm, pred12_hbm, oscore_hbm, oc12_hbm,
                sbuf, hist, ka, va, kb, vb, base, sstage, istage, c12stage,
                sem):
        wid = lax.axis_index("s") * 2 + lax.axis_index("c")

        def one_instance(j, _):
            e = wid * PER_W + j
            b = e // NCLS
            pltpu.sync_copy(scores_hbm.at[e], sbuf)

            for t in range(NBKT // 16):
                hist[pl.ds(t * 16, 16)] = jnp.zeros((16,), i32)

            def hbody(t, carry):
                s = sbuf[pl.ds(t * 16, 16)]
                valid = s > CONF_THRESH
                bkt = jnp.clip((s * 256.0).astype(i32), 0, NBKT - 1)
                cnt, last = plsc.scan_count(bkt, valid)
                plsc.addupdate_scatter(hist, [bkt], cnt, mask=last)
                return carry
            lax.fori_loop(0, NVEC, hbody, 0)

            total = jnp.int32(0)
            for t in range(NBKT // 16):
                total = total + jnp.sum(hist[pl.ds(t * 16, 16)])
            T = jnp.maximum(total - CAND, 0)
            nb = jnp.int32(0)
            running = jnp.int32(0)
            for t in range(NBKT // 16):
                h = hist[pl.ds(t * 16, 16)]
                incl = plsc.cumsum(h)
                excl = running + incl - h
                cond = excl <= T
                nb = nb + jnp.max(plsc.all_reduce_population_count(cond))
                running = running + jnp.sum(h)
            B = nb - 1

            def cbody(t, cnt):
                s = sbuf[pl.ds(t * 16, 16)]
                valid = s > CONF_THRESH
                bkt = jnp.clip((s * 256.0).astype(i32), 0, NBKT - 1)
                m = valid & (bkt >= B)
                key = ~plsc.bitcast(s, u32)
                idx = iota16() + t * 16
                plsc.store_compressed(ka.at[pl.ds(cnt, 16)], key, m)
                plsc.store_compressed(va.at[pl.ds(cnt, 16)], idx, m)
                return cnt + jnp.max(plsc.all_reduce_population_count(m))
            C = lax.fori_loop(0, NVEC, cbody, jnp.int32(0))
            nv = (C + 15) // 16

            for p in range(7):
                src_k, src_v = (ka, va) if p % 2 == 0 else (kb, vb)
                dst_k, dst_v = (kb, vb) if p % 2 == 0 else (ka, va)
                sh = 5 * p
                base[pl.ds(0, 16)] = jnp.zeros((16,), i32)
                base[pl.ds(16, 16)] = jnp.zeros((16,), i32)

                def rhist(t, carry, src_k=src_k, sh=sh):
                    k = src_k[pl.ds(t * 16, 16)]
                    m = (iota16() + t * 16) < C
                    d = (lax.shift_right_logical(k, u32(sh)) & u32(31)).astype(i32)
                    cnt, last = plsc.scan_count(d, m)
                    plsc.addupdate_scatter(base, [d], cnt, mask=last & m)
                    return carry
                lax.fori_loop(0, nv, rhist, 0)

                b0 = base[pl.ds(0, 16)]
                b1 = base[pl.ds(16, 16)]
                e0 = plsc.cumsum(b0) - b0
                s0 = jnp.sum(b0)
                e1 = plsc.cumsum(b1) - b1 + s0
                base[pl.ds(0, 16)] = e0
                base[pl.ds(16, 16)] = e1

                def rperm(t, carry, src_k=src_k, src_v=src_v,
                          dst_k=dst_k, dst_v=dst_v, sh=sh):
                    k = src_k[pl.ds(t * 16, 16)]
                    v = src_v[pl.ds(t * 16, 16)]
                    m = (iota16() + t * 16) < C
                    d = (lax.shift_right_logical(k, u32(sh)) & u32(31)).astype(i32)
                    cnt, last = plsc.scan_count(d, m)
                    off = plsc.load_gather(base, [d]) + cnt - 1
                    plsc.store_scatter(dst_k, [off], k, mask=m)
                    plsc.store_scatter(dst_v, [off], v, mask=m)
                    plsc.addupdate_scatter(base, [d], cnt, mask=last & m)
                    return carry
                lax.fori_loop(0, nv, rperm, 0)

            for t in range(CAND // 16):
                m = (iota16() + t * 16) < jnp.minimum(C, CAND)
                k = kb[pl.ds(t * 16, 16)]
                v = vb[pl.ds(t * 16, 16)]
                sstage[pl.ds(t * 16, 16)] = jnp.where(
                    m, plsc.bitcast(~k, f32), -jnp.inf)
                istage[pl.ds(t * 16, 16)] = jnp.where(m, v + b * N_BOX, 0)

            pltpu.async_copy(pred12_hbm.at[istage], c12stage, sem).wait()
            pltpu.sync_copy(sstage, oscore_hbm.at[e])
            pltpu.sync_copy(c12stage, oc12_hbm.at[e])
            return 0

        lax.fori_loop(0, PER_W, one_instance, 0)

    return sc_body(scoresT, pred12)


@jax.jit
def kernel(prediction):
    B = prediction.shape[0]
    conf = prediction[:, :, 1:1 + NCLS]
    masked = jnp.where(conf > CONF_THRESH, conf, -jnp.inf)
    cand_scores, cand_idx = lax.top_k(masked.transpose(0, 2, 1), CAND)
    cols12 = prediction[:, :, 81:93]
    cand12 = jnp.take_along_axis(cols12[:, None], cand_idx[..., None], axis=2)
    c12_in = cand12.transpose(0, 3, 2, 1)
    s_in = cand_scores.transpose(0, 2, 1)
    return _nms_pallas(s_in, c12_in)
